# Initial kernel scaffold; baseline (speedup 1.0000x reference)
import jax
import jax.numpy as jnp
from jax.experimental import pallas as pl


def kernel(vid2fill, patches, queryInds):
    raise NotImplementedError("write your pallas kernel here")


# speedup vs baseline: 10.3701x; 10.3701x over previous
import functools

import jax
import jax.numpy as jnp
from jax import lax
from jax.experimental import pallas as pl
from jax.experimental.pallas import tpu as pltpu
from jax.experimental.pallas import tpu_sc as plsc

T, C, H, W = 8, 3, 512, 512
PS, PT = 7, 1
PWORDS = PT * C * PS * PS
PW_PAD = 160
NC, NS, L = 2, 16, 16

Q_TOTAL = 131072
NB = NS
BROWS = H // NB
HALO = PS - 1
AROWS = BROWS + HALO
ACC_N = C * AROWS * W
TMOD = T - PT + 1
YMOD = H - PS + 1
XMOD = W - PS + 1
ROUNDS = T // NC

CH = 8192
CAP = CH + 64
GRP = 64


def _word_off(j):
    c = j // (PS * PS)
    r = j - c * (PS * PS)
    dy = r // PS
    dx = r - dy * PS
    return c * (AROWS * W) + dy * W + dx


def _body(vid_h, t_h, y_h, x_h, pat_h, out_h,
          tv, yv, xv, ids, bas, tab, stg, acc, halo_v, shr, sem):
    cid = lax.axis_index("c")
    sid = lax.axis_index("s")
    ystart = sid * BROWS
    lane = lax.iota(jnp.int32, L)

    for jj in range(PW_PAD // L):
        j_v = lane + jj * L
        off = _word_off(j_v)
        tab[pl.ds(jj * L, L)] = jnp.where(j_v < PWORDS, off, 0)

    def round_body(r, _):
        f = cid * ROUNDS + r
        plsc.subcore_barrier()

        for c in range(C):
            pltpu.sync_copy(
                vid_h.at[pl.ds(f * (C * H * W) + c * (H * W) + ystart * W,
                               BROWS * W)],
                acc.at[pl.ds(c * AROWS * W, BROWS * W)])

        def zbody(i, _):
            z = jnp.zeros((L,), jnp.float32)
            for c in range(C):
                acc[pl.ds(c * AROWS * W + BROWS * W + i * L, L)] = z
            return 0
        lax.fori_loop(0, HALO * W // L, zbody, 0)

        def chunk_body(ch, _):
            co = ch * CH
            pltpu.sync_copy(t_h.at[pl.ds(co, CH)], tv)
            pltpu.sync_copy(y_h.at[pl.ds(co, CH)], yv)
            pltpu.sync_copy(x_h.at[pl.ds(co, CH)], xv)

            def fbody(i, cur):
                tvv = tv[pl.ds(i * L, L)]
                yvv = yv[pl.ds(i * L, L)]
                xvv = xv[pl.ds(i * L, L)]
                t0 = jnp.bitwise_and(tvv, TMOD - 1)
                y0 = lax.rem(yvv, YMOD)
                x0 = lax.rem(xvv, XMOD)
                yl = y0 - ystart
                m = (t0 == f) & (yl >= 0) & (yl < BROWS)
                gid = co + i * L + lane
                base = yl * W + x0
                plsc.store_compressed(ids.at[pl.ds(cur, L)], gid, mask=m)
                plsc.store_compressed(bas.at[pl.ds(cur, L)], base, mask=m)
                return cur + jnp.sum(m.astype(jnp.int32))

            n = lax.fori_loop(0, CH // L, fbody, 0)

            z = jnp.zeros((L,), jnp.int32)
            for pp in range(GRP // L):
                ids[pl.ds(n + pp * L, L)] = z
                bas[pl.ds(n + pp * L, L)] = z

            nv = jnp.full((L,), n, jnp.int32)
            ng = (n + GRP - 1) // GRP

            def gbody(g, _):
                k = g * GRP
                pltpu.async_copy(pat_h.at[ids.at[pl.ds(k, GRP)]], stg,
                                 sem).wait()
                for sg in range(GRP // L):
                    kb = k + sg * L
                    for p in range(L):
                        pidx = sg * L + p
                        kpv = jnp.full((L,), kb + p, jnp.int32)
                        vm = kpv < nv
                        bb = plsc.load_gather(bas, [kpv])
                        for jj in range(PW_PAD // L):
                            vals = stg[pidx, pl.ds(jj * L, L)]
                            idxv = tab[pl.ds(jj * L, L)] + bb
                            plsc.addupdate_scatter(acc, [idxv], vals, mask=vm)
                return 0

            lax.fori_loop(0, ng, gbody, 0)
            return 0

        lax.fori_loop(0, (Q_TOTAL // CH), chunk_body, 0)

        for c in range(C):
            pltpu.sync_copy(acc.at[pl.ds(c * AROWS * W + BROWS * W, HALO * W)],
                            shr.at[sid, pl.ds(c * HALO * W, HALO * W)])
        plsc.subcore_barrier()

        @pl.when(sid > 0)
        def _():
            pltpu.sync_copy(shr.at[sid - 1], halo_v)

            def hbody(i, _):
                for c in range(C):
                    plsc.addupdate(
                        acc.at[pl.ds(c * AROWS * W + i * L, L)],
                        halo_v[pl.ds(c * HALO * W + i * L, L)])
                return 0
            lax.fori_loop(0, HALO * W // L, hbody, 0)

        for c in range(C):
            pltpu.sync_copy(
                acc.at[pl.ds(c * AROWS * W, BROWS * W)],
                out_h.at[pl.ds(f * (C * H * W) + c * (H * W) + ystart * W,
                               BROWS * W)])
        return 0

    lax.fori_loop(0, ROUNDS, round_body, 0)


@functools.lru_cache(maxsize=None)
def _build_kernel():
    mesh = plsc.VectorSubcoreMesh(core_axis_name="c", subcore_axis_name="s",
                                  num_cores=NC, num_subcores=NS)
    return pl.kernel(
        _body,
        out_type=jax.ShapeDtypeStruct((T * C * H * W,), jnp.float32),
        mesh=mesh,
        compiler_params=pltpu.CompilerParams(needs_layout_passes=False,
                                             use_tc_tiling_on_sc=False),
        scratch_types=[
            pltpu.VMEM((CH,), jnp.int32),
            pltpu.VMEM((CH,), jnp.int32),
            pltpu.VMEM((CH,), jnp.int32),
            pltpu.VMEM((CAP,), jnp.int32),
            pltpu.VMEM((CAP,), jnp.int32),
            pltpu.VMEM((PW_PAD,), jnp.int32),
            pltpu.VMEM((GRP, PW_PAD), jnp.float32),
            pltpu.VMEM((ACC_N,), jnp.float32),
            pltpu.VMEM((C * HALO * W,), jnp.float32),
            pltpu.VMEM_SHARED((NS, C * HALO * W), jnp.float32),
            pltpu.SemaphoreType.DMA,
        ],
    )


def kernel(vid2fill, patches, queryInds):
    q = queryInds.astype(jnp.int32)
    t = q[:, 0]
    y = q[:, 1]
    x = q[:, 2]
    pat = jnp.pad(patches.reshape(patches.shape[0], PWORDS),
                  ((0, 0), (0, PW_PAD - PWORDS)))
    vidf = vid2fill.reshape(-1)
    out = _build_kernel()(vidf, t, y, x, pat)
    return out.reshape(T, C, H, W)


# speedup vs baseline: 13.5851x; 1.3100x over previous
import functools

import jax
import jax.numpy as jnp
from jax import lax
from jax.experimental import pallas as pl
from jax.experimental.pallas import tpu as pltpu
from jax.experimental.pallas import tpu_sc as plsc

T, C, H, W = 8, 3, 512, 512
PS, PT = 7, 1
PWORDS = PT * C * PS * PS
PW_PAD = 160
NC, NS, L = 2, 16, 16

Q_TOTAL = 131072
NB = NS
BROWS = H // NB
HALO = PS - 1
AROWS = BROWS + HALO
ACC_N = C * AROWS * W
TMOD = T - PT + 1
YMOD = H - PS + 1
XMOD = W - PS + 1
ROUNDS = T // NC

CH = 8192
CHA = 4096
CAP = CH + 64
GRP = 64


def _word_off(j):
    c = j // (PS * PS)
    r = j - c * (PS * PS)
    dy = r // PS
    dx = r - dy * PS
    return c * (AROWS * W) + dy * W + dx


def _body(vid_h, t_h, y_h, x_h, pat_h, out_h,
          pkb, ids, bas, tab, stg0, stg1, acc, halo_v, shr, pks,
          sem0, sem1):
    cid = lax.axis_index("c")
    sid = lax.axis_index("s")
    ystart = sid * BROWS
    lane = lax.iota(jnp.int32, L)

    for jj in range(PW_PAD // L):
        j_v = lane + jj * L
        off = _word_off(j_v)
        tab[pl.ds(jj * L, L)] = jnp.where(j_v < PWORDS, off, 0)

    for a in range(Q_TOTAL // NS // CHA):
        off = sid * (Q_TOTAL // NS) + a * CHA
        pltpu.sync_copy(t_h.at[pl.ds(off, CHA)], ids.at[pl.ds(0, CHA)])
        pltpu.sync_copy(y_h.at[pl.ds(off, CHA)], bas.at[pl.ds(0, CHA)])
        pltpu.sync_copy(x_h.at[pl.ds(off, CHA)], pkb.at[pl.ds(0, CHA)])

        def abody(i, _):
            tvv = ids[pl.ds(i * L, L)]
            yvv = bas[pl.ds(i * L, L)]
            xvv = pkb[pl.ds(i * L, L)]
            t0 = jnp.bitwise_and(tvv, TMOD - 1)
            y0 = lax.rem(yvv, YMOD)
            x0 = lax.rem(xvv, XMOD)
            band = y0 // BROWS
            base = (jnp.bitwise_and(y0, BROWS - 1) << 9) + x0
            packed = (((t0 << 4) | band) << 14) | base
            pkb[pl.ds(CHA + i * L, L)] = packed
            return 0
        lax.fori_loop(0, CHA // L, abody, 0)
        pltpu.sync_copy(pkb.at[pl.ds(CHA, CHA)], pks.at[pl.ds(off, CHA)])

    def round_body(r, _):
        f = cid * ROUNDS + r
        plsc.subcore_barrier()

        for c in range(C):
            pltpu.sync_copy(
                vid_h.at[pl.ds(f * (C * H * W) + c * (H * W) + ystart * W,
                               BROWS * W)],
                acc.at[pl.ds(c * AROWS * W, BROWS * W)])

        def zbody(i, _):
            z = jnp.zeros((L,), jnp.float32)
            for c in range(C):
                acc[pl.ds(c * AROWS * W + BROWS * W + i * L, L)] = z
            return 0
        lax.fori_loop(0, HALO * W // L, zbody, 0)

        want = (f << 4) + sid

        def chunk_body(ch, _):
            co = ch * CH
            pltpu.sync_copy(pks.at[pl.ds(co, CH)], pkb)

            def fbody(i, cur):
                pv = pkb[pl.ds(i * L, L)]
                m = (pv >> 14) == want
                base = jnp.bitwise_and(pv, 16383)
                gid = co + i * L + lane
                plsc.store_compressed(ids.at[pl.ds(cur, L)], gid, mask=m)
                plsc.store_compressed(bas.at[pl.ds(cur, L)], base, mask=m)
                return cur + plsc.all_reduce_population_count(m)[0]

            n = lax.fori_loop(0, CH // L, fbody, 0)

            z = jnp.zeros((L,), jnp.int32)
            for pp in range(GRP // L):
                ids[pl.ds(n + pp * L, L)] = z
                bas[pl.ds(n + pp * L, L)] = z

            ngf = n // GRP

            def issue(g, stg, sem):
                pltpu.async_copy(pat_h.at[ids.at[pl.ds(g * GRP, GRP)]],
                                 stg, sem)

            def wait(stg, sem):
                pltpu.make_async_copy(pat_h.at[ids.at[pl.ds(0, GRP)]],
                                      stg, sem).wait()

            def proc_full(stg, k):
                def sgbody(sg, _):
                    kb = k + sg * L
                    for p in range(L):
                        kpv = jnp.full((L,), kb + p, jnp.int32)
                        bb = plsc.load_gather(bas, [kpv])
                        row = sg * L + p
                        for jj in range(PW_PAD // L):
                            vals = stg[row, pl.ds(jj * L, L)]
                            idxv = tab[pl.ds(jj * L, L)] + bb
                            plsc.addupdate_scatter(acc, [idxv], vals)
                    return 0
                lax.fori_loop(0, GRP // L, sgbody, 0)

            @pl.when(ngf > 0)
            def _():
                issue(0, stg0, sem0)

            @pl.when(ngf > 1)
            def _():
                issue(1, stg1, sem1)

            def pair_body(q, _):
                g0 = 2 * q
                g1 = g0 + 1

                @pl.when(g0 < ngf)
                def _():
                    wait(stg0, sem0)
                    proc_full(stg0, g0 * GRP)

                    @pl.when(g0 + 2 < ngf)
                    def _():
                        issue(g0 + 2, stg0, sem0)

                @pl.when(g1 < ngf)
                def _():
                    wait(stg1, sem1)
                    proc_full(stg1, g1 * GRP)

                    @pl.when(g1 + 2 < ngf)
                    def _():
                        issue(g1 + 2, stg1, sem1)
                return 0

            lax.fori_loop(0, (ngf + 1) // 2, pair_body, 0)

            rem_n = n - ngf * GRP

            @pl.when(rem_n > 0)
            def _():
                k = ngf * GRP
                pltpu.async_copy(pat_h.at[ids.at[pl.ds(k, GRP)]], stg0,
                                 sem0).wait()
                nv = jnp.full((L,), n, jnp.int32)

                def sgbody(sg, _):
                    kb = k + sg * L
                    for p in range(L):
                        kpv = jnp.full((L,), kb + p, jnp.int32)
                        vm = kpv < nv
                        bb = plsc.load_gather(bas, [kpv])
                        row = sg * L + p
                        for jj in range(PW_PAD // L):
                            vals = stg0[row, pl.ds(jj * L, L)]
                            idxv = tab[pl.ds(jj * L, L)] + bb
                            plsc.addupdate_scatter(acc, [idxv], vals,
                                                   mask=vm)
                    return 0
                lax.fori_loop(0, GRP // L, sgbody, 0)
            return 0

        lax.fori_loop(0, (Q_TOTAL // CH), chunk_body, 0)

        for c in range(C):
            pltpu.sync_copy(acc.at[pl.ds(c * AROWS * W + BROWS * W, HALO * W)],
                            shr.at[sid, pl.ds(c * HALO * W, HALO * W)])
        plsc.subcore_barrier()

        @pl.when(sid > 0)
        def _():
            pltpu.sync_copy(shr.at[sid - 1], halo_v)

            def hbody(i, _):
                for c in range(C):
                    plsc.addupdate(
                        acc.at[pl.ds(c * AROWS * W + i * L, L)],
                        halo_v[pl.ds(c * HALO * W + i * L, L)])
                return 0
            lax.fori_loop(0, HALO * W // L, hbody, 0)

        for c in range(C):
            pltpu.sync_copy(
                acc.at[pl.ds(c * AROWS * W, BROWS * W)],
                out_h.at[pl.ds(f * (C * H * W) + c * (H * W) + ystart * W,
                               BROWS * W)])
        return 0

    lax.fori_loop(0, ROUNDS, round_body, 0)


@functools.lru_cache(maxsize=None)
def _build_kernel():
    mesh = plsc.VectorSubcoreMesh(core_axis_name="c", subcore_axis_name="s",
                                  num_cores=NC, num_subcores=NS)
    return pl.kernel(
        _body,
        out_type=jax.ShapeDtypeStruct((T * C * H * W,), jnp.float32),
        mesh=mesh,
        compiler_params=pltpu.CompilerParams(needs_layout_passes=False,
                                             use_tc_tiling_on_sc=False),
        scratch_types=[
            pltpu.VMEM((CH,), jnp.int32),
            pltpu.VMEM((CAP,), jnp.int32),
            pltpu.VMEM((CAP,), jnp.int32),
            pltpu.VMEM((PW_PAD,), jnp.int32),
            pltpu.VMEM((GRP, PW_PAD), jnp.float32),
            pltpu.VMEM((GRP, PW_PAD), jnp.float32),
            pltpu.VMEM((ACC_N,), jnp.float32),
            pltpu.VMEM((C * HALO * W,), jnp.float32),
            pltpu.VMEM_SHARED((NS, C * HALO * W), jnp.float32),
            pltpu.VMEM_SHARED((Q_TOTAL,), jnp.int32),
            pltpu.SemaphoreType.DMA,
            pltpu.SemaphoreType.DMA,
        ],
    )


def kernel(vid2fill, patches, queryInds):
    q = queryInds.astype(jnp.int32)
    t = q[:, 0]
    y = q[:, 1]
    x = q[:, 2]
    pat = jnp.pad(patches.reshape(patches.shape[0], PWORDS),
                  ((0, 0), (0, PW_PAD - PWORDS)))
    vidf = vid2fill.reshape(-1)
    out = _build_kernel()(vidf, t, y, x, pat)
    return out.reshape(T, C, H, W)
